# Initial kernel scaffold; baseline (speedup 1.0000x reference)
import jax
import jax.numpy as jnp
from jax.experimental import pallas as pl


def kernel(input_ids, input_mask, embedding_table):
    raise NotImplementedError("write your pallas kernel here")


# speedup vs baseline: 1.9776x; 1.9776x over previous
import functools

import jax
import jax.numpy as jnp
from jax import lax
from jax.experimental import pallas as pl
from jax.experimental.pallas import tpu as pltpu
from jax.experimental.pallas import tpu_sc as plsc

_D = 64
_B = 16384
_H = 50
_N = _B * _H
_NC = 2
_NS = 16
_L = 16
_NW = _NC * _NS
_PER_W = _N // _NW
_CHUNK = 512
_NCHUNK = _PER_W // _CHUNK
_GROUPS = _CHUNK // _L


def _tec_body(ids_hbm, mask_hbm, table_hbm, out_hbm,
              ids_v, mask_v, gidx_v, rows_v, sem):
    wid = lax.axis_index("s") * _NC + lax.axis_index("c")
    base = wid * _PER_W

    def chunk_body(c, carry):
        off = base + c * _CHUNK
        pltpu.sync_copy(ids_hbm.at[pl.ds(off, _CHUNK)], ids_v)
        pltpu.sync_copy(mask_hbm.at[pl.ds(off, _CHUNK)], mask_v)

        def prelude(g, carry2):
            s = pl.ds(g * _L, _L)
            gidx_v[s] = ids_v[s] * mask_v[s]
            return carry2

        lax.fori_loop(0, _GROUPS, prelude, 0)

        pltpu.async_copy(table_hbm.at[gidx_v], rows_v, sem).wait()

        def group(g, carry2):
            mvec = mask_v[pl.ds(g * _L, _L)].astype(jnp.float32)
            for l in range(_L):
                r = g * _L + l
                m = lax.gather(
                    mvec, jnp.full((_L, 1), l, jnp.int32),
                    lax.GatherDimensionNumbers(
                        offset_dims=(), collapsed_slice_dims=(0,),
                        start_index_map=(0,)),
                    (1,), mode=lax.GatherScatterMode.PROMISE_IN_BOUNDS)
                for j in range(_D // _L):
                    s = pl.ds(j * _L, _L)
                    rows_v[r, s] = rows_v[r, s] * m
            return carry2

        lax.fori_loop(0, _GROUPS, group, 0)

        pltpu.sync_copy(rows_v, out_hbm.at[pl.ds(off, _CHUNK)])
        return carry

    lax.fori_loop(0, _NCHUNK, chunk_body, 0)


@jax.jit
def kernel(input_ids, input_mask, embedding_table):
    ids = input_ids.reshape(-1).astype(jnp.int32)
    mask = input_mask.reshape(-1).astype(jnp.int32)
    mesh = plsc.VectorSubcoreMesh(core_axis_name="c", subcore_axis_name="s")
    call = pl.kernel(
        _tec_body,
        mesh=mesh,
        compiler_params=pltpu.CompilerParams(use_tc_tiling_on_sc=False),
        out_type=jax.ShapeDtypeStruct((_N, _D), jnp.float32),
        scratch_types=[
            pltpu.VMEM((_CHUNK,), jnp.int32),
            pltpu.VMEM((_CHUNK,), jnp.int32),
            pltpu.VMEM((_CHUNK,), jnp.int32),
            pltpu.VMEM((_CHUNK, _D), jnp.float32),
            pltpu.SemaphoreType.DMA,
        ],
    )
    out = call(ids, mask, embedding_table)
    return out.reshape(_B, _H, _D)
